# baseline (speedup 1.0000x reference)
import jax
import jax.numpy as jnp
from jax.experimental import pallas as pl


def _bcast_add_kernel(w_ref, xs_ref, o_ref):
    w = w_ref[...]
    xs = xs_ref[...]
    o_ref[...] = w[:, None, :] + xs[None, :, :]


def kernel(x, pos_embed_weight):
    seq_len, batch_size = x.shape
    _, dim = pos_embed_weight.shape
    LANES = 128
    rep = LANES // dim
    sb = seq_len // rep

    wlane = jnp.tile(pos_embed_weight[:seq_len], (1, rep))
    xs = jnp.repeat(x[:, 0], dim).reshape(sb, LANES)

    BI = 64
    out3 = pl.pallas_call(
        _bcast_add_kernel,
        grid=(seq_len // BI,),
        in_specs=[
            pl.BlockSpec((BI, LANES), lambda i: (i, 0)),
            pl.BlockSpec((sb, LANES), lambda i: (0, 0)),
        ],
        out_specs=pl.BlockSpec((BI, sb, LANES), lambda i: (i, 0, 0)),
        out_shape=jax.ShapeDtypeStruct((seq_len, sb, LANES), jnp.float32),
    )(wlane, xs)

    return out3.reshape(seq_len, seq_len, dim)
